# baseline (speedup 1.0000x reference)
import functools

import jax
import jax.numpy as jnp
from jax import lax
from jax.experimental import pallas as pl
from jax.experimental.pallas import tpu as pltpu
from jax.experimental.pallas import tpu_sc as plsc

_BATCH = 16384
_EMB = 64
_BACKEND = 128
_NCLS = 128
_PACK = 2 * _EMB

_CH = 128


def _build_gather(batch, half_rows):
    info = plsc.get_sparse_core_info()
    nw = info.num_cores * info.num_subcores
    b_per_w = batch // nw
    n_ch = b_per_w // _CH
    mesh = plsc.VectorSubcoreMesh(core_axis_name="c", subcore_axis_name="s")

    @functools.partial(
        pl.kernel,
        mesh=mesh,
        out_type=jax.ShapeDtypeStruct((batch, _PACK), jnp.float32),
        scratch_types=[
            pltpu.VMEM((b_per_w,), jnp.int32),
            pltpu.VMEM((_CH, _PACK), jnp.float32),
            pltpu.SemaphoreType.DMA,
        ],
    )
    def gather(idx_hbm, table_hbm, out_hbm, idx_v, rows_v, sem):
        wid = lax.axis_index("s") * info.num_cores + lax.axis_index("c")
        base = wid * b_per_w
        pltpu.sync_copy(idx_hbm.at[pl.ds(base, b_per_w)], idx_v)
        for j in range(n_ch):
            pltpu.async_copy(
                table_hbm.at[idx_v.at[pl.ds(j * _CH, _CH)]], rows_v, sem
            ).wait()
            pltpu.sync_copy(rows_v, out_hbm.at[pl.ds(base + j * _CH, _CH)])

    return gather


_gather = _build_gather(_BATCH, 50000)


def _mm_body(x_ref, pk_ref, par_ref, w1_ref, w2s_ref, b_ref, o_ref):
    p = par_ref[...]
    col = lax.broadcasted_iota(jnp.int32, pk_ref.shape, 1)
    mask = jnp.where(col < _EMB, 1.0 - p, p)
    o_ref[...] = (
        jnp.dot(x_ref[...], w1_ref[...], preferred_element_type=jnp.float32)
        + jnp.dot(pk_ref[...] * mask, w2s_ref[...],
                  preferred_element_type=jnp.float32)
        + b_ref[...]
    )


def kernel(batched_data, class_emb, W, b, domains):
    packed = class_emb.reshape(50000, _PACK)
    idx_half = domains >> 1
    parity = (domains & 1).astype(jnp.float32).reshape(_BATCH, 1)

    feat_packed = _gather(idx_half, packed)

    w1t = W[:, :_BACKEND].T
    w2t = W[:, _BACKEND:].T
    w2s = jnp.concatenate([w2t, w2t], axis=0)
    b2d = b.reshape(1, _NCLS)

    bm = 2048
    out = pl.pallas_call(
        _mm_body,
        grid=(_BATCH // bm,),
        in_specs=[
            pl.BlockSpec((bm, _BACKEND), lambda i: (i, 0)),
            pl.BlockSpec((bm, _PACK), lambda i: (i, 0)),
            pl.BlockSpec((bm, 1), lambda i: (i, 0)),
            pl.BlockSpec((_BACKEND, _NCLS), lambda i: (0, 0)),
            pl.BlockSpec((_PACK, _NCLS), lambda i: (0, 0)),
            pl.BlockSpec((1, _NCLS), lambda i: (0, 0)),
        ],
        out_specs=pl.BlockSpec((bm, _NCLS), lambda i: (i, 0)),
        out_shape=jax.ShapeDtypeStruct((_BATCH, _NCLS), jnp.float32),
    )(batched_data, feat_packed, parity, w1t, w2s, b2d)
    return out


# speedup vs baseline: 1.8279x; 1.8279x over previous
import functools

import jax
import jax.numpy as jnp
from jax import lax
from jax.experimental import pallas as pl
from jax.experimental.pallas import tpu as pltpu
from jax.experimental.pallas import tpu_sc as plsc

_BATCH = 16384
_EMB = 64
_BACKEND = 128
_NCLS = 128
_VOCAB = 100000

_HALF = 8192
_UNROLL = 8


def _build_gather_t():
    info = plsc.get_sparse_core_info()
    nw = info.num_cores * info.num_subcores
    rounds = _EMB // nw
    n_half = _BATCH // _HALF
    groups = _HALF // 16
    mesh = plsc.VectorSubcoreMesh(core_axis_name="c", subcore_axis_name="s")

    @functools.partial(
        pl.kernel,
        mesh=mesh,
        compiler_params=pltpu.CompilerParams(needs_layout_passes=False),
        out_type=jax.ShapeDtypeStruct((_EMB, _BATCH), jnp.float32),
        scratch_types=[
            pltpu.VMEM((_VOCAB,), jnp.float32),
            pltpu.VMEM((_HALF,), jnp.int32),
            pltpu.VMEM((_HALF,), jnp.float32),
        ],
    )
    def gather_t(idx_hbm, table_t_hbm, out_hbm, row_v, idx_v, out_v):
        wid = lax.axis_index("s") * info.num_cores + lax.axis_index("c")
        for r in range(rounds):
            f = wid * rounds + r
            pltpu.sync_copy(table_t_hbm.at[f], row_v)
            for h in range(n_half):
                pltpu.sync_copy(idx_hbm.at[pl.ds(h * _HALF, _HALF)], idx_v)

                def body(g, _):
                    base = g * (16 * _UNROLL)
                    for u in range(_UNROLL):
                        sl = pl.ds(base + u * 16, 16)
                        out_v[sl] = plsc.load_gather(row_v, [idx_v[sl]])
                    return 0

                lax.fori_loop(0, groups // _UNROLL, body, 0)
                pltpu.sync_copy(out_v, out_hbm.at[f, pl.ds(h * _HALF, _HALF)])

    return gather_t


_gather_t = _build_gather_t()


def _mm_body(x_ref, ft_ref, w1_ref, w2_ref, b_ref, o_ref):
    o_ref[...] = (
        jnp.dot(x_ref[...], w1_ref[...], preferred_element_type=jnp.float32)
        + lax.dot_general(
            ft_ref[...], w2_ref[...], (((0,), (0,)), ((), ())),
            preferred_element_type=jnp.float32)
        + b_ref[...]
    )


def kernel(batched_data, class_emb, W, b, domains):
    table_t = class_emb.T
    feat_t = _gather_t(domains, table_t)

    w1t = W[:, :_BACKEND].T
    w2t = W[:, _BACKEND:].T
    b2d = b.reshape(1, _NCLS)

    bm = 2048
    out = pl.pallas_call(
        _mm_body,
        grid=(_BATCH // bm,),
        in_specs=[
            pl.BlockSpec((bm, _BACKEND), lambda i: (i, 0)),
            pl.BlockSpec((_EMB, bm), lambda i: (0, i)),
            pl.BlockSpec((_BACKEND, _NCLS), lambda i: (0, 0)),
            pl.BlockSpec((_EMB, _NCLS), lambda i: (0, 0)),
            pl.BlockSpec((1, _NCLS), lambda i: (0, 0)),
        ],
        out_specs=pl.BlockSpec((bm, _NCLS), lambda i: (i, 0)),
        out_shape=jax.ShapeDtypeStruct((_BATCH, _NCLS), jnp.float32),
    )(batched_data, feat_t, w1t, w2t, b2d)
    return out


# speedup vs baseline: 1.8873x; 1.0325x over previous
import functools

import jax
import jax.numpy as jnp
from jax import lax
from jax.experimental import pallas as pl
from jax.experimental.pallas import tpu as pltpu
from jax.experimental.pallas import tpu_sc as plsc

_BATCH = 16384
_EMB = 64
_BACKEND = 128
_NCLS = 128
_VOCAB = 100000

_HALF = 8192
_UNROLL = 8


def _build_gather_t():
    info = plsc.get_sparse_core_info()
    nw = info.num_cores * info.num_subcores
    rounds = _EMB // nw
    n_half = _BATCH // _HALF
    groups = _HALF // 16
    mesh = plsc.VectorSubcoreMesh(core_axis_name="c", subcore_axis_name="s")

    @functools.partial(
        pl.kernel,
        mesh=mesh,
        compiler_params=pltpu.CompilerParams(needs_layout_passes=False),
        out_type=jax.ShapeDtypeStruct((_EMB, _BATCH), jnp.float32),
        scratch_types=[
            pltpu.VMEM((_VOCAB,), jnp.float32),
            pltpu.VMEM((_HALF,), jnp.int32),
            pltpu.VMEM((_BATCH,), jnp.float32),
        ],
    )
    def gather_t(idx_hbm, table_t_hbm, out_hbm, row_v, idx_v, out_v):
        wid = lax.axis_index("s") * info.num_cores + lax.axis_index("c")

        def row_body(r, _):
            f = wid * rounds + r
            pltpu.sync_copy(table_t_hbm.at[f], row_v)

            def half_body(h, _):
                pltpu.sync_copy(idx_hbm.at[pl.ds(h * _HALF, _HALF)], idx_v)

                def body(g, _):
                    base = g * (16 * _UNROLL)
                    for u in range(_UNROLL):
                        out_v[pl.ds(h * _HALF + base + u * 16, 16)] = (
                            plsc.load_gather(row_v, [idx_v[pl.ds(base + u * 16, 16)]])
                        )
                    return 0

                lax.fori_loop(0, groups // _UNROLL, body, 0)
                return 0

            lax.fori_loop(0, n_half, half_body, 0)
            pltpu.sync_copy(out_v, out_hbm.at[f])
            return 0

        lax.fori_loop(0, rounds, row_body, 0)

    return gather_t


_gather_t = _build_gather_t()


def _mm_body(x_ref, ft_ref, w1_ref, w2_ref, b_ref, o_ref):
    o_ref[...] = (
        jnp.dot(x_ref[...], w1_ref[...], preferred_element_type=jnp.float32)
        + lax.dot_general(
            ft_ref[...], w2_ref[...], (((0,), (0,)), ((), ())),
            preferred_element_type=jnp.float32)
        + b_ref[...]
    )


def kernel(batched_data, class_emb, W, b, domains):
    table_t = class_emb.T
    feat_t = _gather_t(domains, table_t)

    w1t = W[:, :_BACKEND].T
    w2t = W[:, _BACKEND:].T
    b2d = b.reshape(1, _NCLS)

    bm = 2048
    out = pl.pallas_call(
        _mm_body,
        grid=(_BATCH // bm,),
        in_specs=[
            pl.BlockSpec((bm, _BACKEND), lambda i: (i, 0)),
            pl.BlockSpec((_EMB, bm), lambda i: (0, i)),
            pl.BlockSpec((_BACKEND, _NCLS), lambda i: (0, 0)),
            pl.BlockSpec((_EMB, _NCLS), lambda i: (0, 0)),
            pl.BlockSpec((1, _NCLS), lambda i: (0, 0)),
        ],
        out_specs=pl.BlockSpec((bm, _NCLS), lambda i: (i, 0)),
        out_shape=jax.ShapeDtypeStruct((_BATCH, _NCLS), jnp.float32),
    )(batched_data, feat_t, w1t, w2t, b2d)
    return out
